# Initial kernel scaffold; baseline (speedup 1.0000x reference)
import jax
import jax.numpy as jnp
from jax.experimental import pallas as pl


def kernel(embs, step):
    raise NotImplementedError("write your pallas kernel here")


# speedup vs baseline: 4.7277x; 4.7277x over previous
import functools
import math

import jax
import jax.numpy as jnp
from jax import lax
from jax.experimental import pallas as pl
from jax.experimental.pallas import tpu as pltpu
from jax.experimental.pallas import tpu_sc as plsc

_MAX_LENGTH = 8192
_EMBED_DIM = 64
_IDX_W = 128
_G = 4


def _gather_kernel(B):
    info = plsc.get_sparse_core_info()
    NW = info.num_cores * info.num_subcores
    n_rows = B // _IDX_W
    rows_per_w = n_rows // NW
    chunks_per_w = rows_per_w // _G
    C = _G * _IDX_W

    mesh = plsc.VectorSubcoreMesh(core_axis_name="c", subcore_axis_name="s")

    @functools.partial(
        pl.kernel,
        mesh=mesh,
        out_type=jax.ShapeDtypeStruct((B, _EMBED_DIM), jnp.float32),
        scratch_types=[
            pltpu.VMEM((_G, _IDX_W), jnp.int32),
            pltpu.VMEM((C, _EMBED_DIM), jnp.float32),
            pltpu.SemaphoreType.DMA,
        ],
        compiler_params=pltpu.CompilerParams(use_tc_tiling_on_sc=False),
    )
    def k(table_hbm, idx_hbm, out_hbm, idx_v, rows_v, sem):
        wid = lax.axis_index("s") * info.num_cores + lax.axis_index("c")
        row_base = wid * rows_per_w

        def body(g, carry):
            rb = row_base + g * _G
            pltpu.sync_copy(idx_hbm.at[pl.ds(rb, _G)], idx_v)
            for j in range(_G):
                pltpu.async_copy(
                    table_hbm.at[idx_v.at[j]],
                    rows_v.at[pl.ds(j * _IDX_W, _IDX_W)],
                    sem,
                )
            for j in range(_G):
                pltpu.make_async_copy(
                    table_hbm.at[idx_v.at[j]],
                    rows_v.at[pl.ds(j * _IDX_W, _IDX_W)],
                    sem,
                ).wait()
            pltpu.sync_copy(rows_v, out_hbm.at[pl.ds(rb * _IDX_W, C)])
            return carry

        lax.fori_loop(0, chunks_per_w, body, 0)

    return k


def kernel(embs, step):
    b, h = step.shape
    B = b * h
    idx = step.reshape(B // _IDX_W, _IDX_W).astype(jnp.int32)
    out = _gather_kernel(B)(embs, idx)
    return out.reshape(b, h, _EMBED_DIM)


# speedup vs baseline: 5.1175x; 1.0825x over previous
import functools

import jax
import jax.numpy as jnp
from jax import lax
from jax.experimental import pallas as pl
from jax.experimental.pallas import tpu as pltpu
from jax.experimental.pallas import tpu_sc as plsc

_EMBED_DIM = 64
_IDX_W = 128
_G = 4
_NBUF = 2


def _gather_kernel(B):
    info = plsc.get_sparse_core_info()
    NW = info.num_cores * info.num_subcores
    n_rows = B // _IDX_W
    rows_per_w = n_rows // NW
    chunks_per_w = rows_per_w // _G
    rounds = chunks_per_w // _NBUF
    C = _G * _IDX_W

    mesh = plsc.VectorSubcoreMesh(core_axis_name="c", subcore_axis_name="s")

    scratch = (
        [pltpu.VMEM((_G, _IDX_W), jnp.int32) for _ in range(_NBUF)]
        + [pltpu.VMEM((C, _EMBED_DIM), jnp.float32) for _ in range(_NBUF)]
        + [pltpu.SemaphoreType.DMA for _ in range(2 * _NBUF)]
    )

    @functools.partial(
        pl.kernel,
        mesh=mesh,
        out_type=jax.ShapeDtypeStruct((B, _EMBED_DIM), jnp.float32),
        scratch_types=scratch,
        compiler_params=pltpu.CompilerParams(use_tc_tiling_on_sc=False),
    )
    def k(table_hbm, idx_hbm, out_hbm, *bufs):
        idx_v = bufs[:_NBUF]
        rows_v = bufs[_NBUF:2 * _NBUF]
        sem_g = bufs[2 * _NBUF:3 * _NBUF]
        sem_w = bufs[3 * _NBUF:]

        wid = lax.axis_index("s") * info.num_cores + lax.axis_index("c")
        row_base = wid * rows_per_w

        def fire_gather(b, chunk):
            pltpu.sync_copy(idx_hbm.at[pl.ds(row_base + chunk * _G, _G)],
                            idx_v[b])
            for j in range(_G):
                pltpu.async_copy(
                    table_hbm.at[idx_v[b].at[j]],
                    rows_v[b].at[pl.ds(j * _IDX_W, _IDX_W)],
                    sem_g[b],
                )

        def wait_gather(b):
            for j in range(_G):
                pltpu.make_async_copy(
                    table_hbm.at[idx_v[b].at[j]],
                    rows_v[b].at[pl.ds(j * _IDX_W, _IDX_W)],
                    sem_g[b],
                ).wait()

        def out_ref(b, chunk):
            return out_hbm.at[pl.ds((row_base + chunk * _G) * _IDX_W, C)]

        def fire_write(b, chunk):
            pltpu.async_copy(rows_v[b], out_ref(b, chunk), sem_w[b])

        def wait_write(b, chunk):
            pltpu.make_async_copy(rows_v[b], out_ref(b, chunk),
                                  sem_w[b]).wait()

        for b in range(_NBUF):
            fire_gather(b, b)

        def body(r, carry):
            c0 = r * _NBUF
            for b in range(_NBUF):
                wait_gather(b)
                fire_write(b, c0 + b)
            for b in range(_NBUF):
                wait_write(b, c0 + b)
                fire_gather(b, c0 + _NBUF + b)
            return carry

        lax.fori_loop(0, rounds - 1, body, 0)

        c0 = (rounds - 1) * _NBUF
        for b in range(_NBUF):
            wait_gather(b)
            fire_write(b, c0 + b)
        for b in range(_NBUF):
            wait_write(b, c0 + b)

    return k


def kernel(embs, step):
    b, h = step.shape
    B = b * h
    idx = step.reshape(B // _IDX_W, _IDX_W).astype(jnp.int32)
    out = _gather_kernel(B)(embs, idx)
    return out.reshape(b, h, _EMBED_DIM)
